# Initial kernel scaffold; baseline (speedup 1.0000x reference)
import jax
import jax.numpy as jnp
from jax.experimental import pallas as pl


def kernel(x, edge_index_forward, edge_index_onset, edge_index_sustain, edge_index_rest, W1_in, b1_in, W1_out, b1_out, W1_lin, b1_lin, W2_in, b2_in, W2_out, b2_out, W2_lin, b2_lin, ln1_g, ln1_b, ln2_g, ln2_b):
    raise NotImplementedError("write your pallas kernel here")


# speedup vs baseline: 4.0221x; 4.0221x over previous
import functools

import jax
import jax.numpy as jnp
from jax import lax
from jax.experimental import pallas as pl
from jax.experimental.pallas import tpu as pltpu
from jax.experimental.pallas import tpu_sc as plsc

N = 10000
E = 160000
DH = 512
LANE = 128
NT = 16
NCORE = 2
EPT = E // NT
B = 100
NB = EPT // B
RPT = 624
RPT_LAST = N - (NT - 1) * RPT


def _sc_aggregate(num_chunks):
    cpc = num_chunks // NCORE

    def body(x_hbm, g_hbm, s_hbm, z_hbm, out_hbm,
             accum, gbuf, sbuf, rows, sem):
        cid = lax.axis_index("c")
        sid = lax.axis_index("s")
        row0 = sid * RPT

        last = sid == NT - 1
        for cc in range(cpc):
            chunk = cid * cpc + cc
            for j in range(8):
                @pl.when(jnp.logical_not(last))
                def _():
                    pltpu.sync_copy(z_hbm.at[pl.ds(0, RPT)],
                                    accum.at[pl.ds(row0, RPT)])

                @pl.when(last)
                def _():
                    pltpu.sync_copy(z_hbm, accum.at[pl.ds(row0, RPT_LAST)])

                pltpu.sync_copy(g_hbm.at[j, sid], gbuf)
                pltpu.sync_copy(s_hbm.at[j, sid], sbuf)
                plsc.subcore_barrier()

                @pl.loop(0, NB)
                def _(b):
                    pltpu.async_copy(
                        x_hbm.at[chunk].at[gbuf.at[b]], rows, sem).wait()
                    pltpu.sync_copy(rows, accum.at[sbuf.at[b]], add=True)

                plsc.subcore_barrier()

                @pl.when(jnp.logical_not(last))
                def _():
                    pltpu.sync_copy(
                        accum.at[pl.ds(row0, RPT)],
                        out_hbm.at[j, chunk, pl.ds(row0, RPT)])

                @pl.when(last)
                def _():
                    pltpu.sync_copy(
                        accum.at[pl.ds(row0, RPT_LAST)],
                        out_hbm.at[j, chunk, pl.ds(row0, RPT_LAST)])

    mesh = plsc.VectorSubcoreMesh(core_axis_name="c", subcore_axis_name="s")
    return pl.kernel(
        body,
        out_type=jax.ShapeDtypeStruct((8, num_chunks, N, LANE), jnp.float32),
        mesh=mesh,
        scratch_types=[
            pltpu.VMEM_SHARED((N, LANE), jnp.float32),
            pltpu.VMEM((NB, B), jnp.int32),
            pltpu.VMEM((NB, B), jnp.int32),
            pltpu.VMEM((B, LANE), jnp.float32),
            pltpu.SemaphoreType.DMA,
        ],
    )


def _tc_matmul_ln(num_chunks, m_blk=2000):
    K = 9 * num_chunks
    grid = (N // m_blk, K)

    def body(agg_ref, x_ref, w_ref, bt_ref, g_ref, b_ref, out_ref, acc_ref):
        k = pl.program_id(1)

        @pl.when(k == 0)
        def _():
            acc_ref[...] = jnp.zeros_like(acc_ref)

        piece = jnp.where(k < 8 * num_chunks, agg_ref[0, 0], x_ref[0])
        acc_ref[...] += jax.lax.dot(
            piece, w_ref[0], preferred_element_type=jnp.float32)

        @pl.when(k == K - 1)
        def _():
            y = acc_ref[...] + bt_ref[...]
            mu = jnp.mean(y, axis=-1, keepdims=True)
            var = jnp.mean((y - mu) ** 2, axis=-1, keepdims=True)
            out_ref[...] = ((y - mu) * jax.lax.rsqrt(var + 1e-5)
                            * g_ref[...] + b_ref[...])

    return pl.pallas_call(
        body,
        grid=grid,
        in_specs=[
            pl.BlockSpec((1, 1, m_blk, LANE),
                         lambda m, k: (jnp.minimum(k // num_chunks, 7),
                                       k % num_chunks, m, 0)),
            pl.BlockSpec((1, m_blk, LANE), lambda m, k: (k % num_chunks, m, 0)),
            pl.BlockSpec((1, LANE, DH), lambda m, k: (k, 0, 0)),
            pl.BlockSpec((1, DH), lambda m, k: (0, 0)),
            pl.BlockSpec((1, DH), lambda m, k: (0, 0)),
            pl.BlockSpec((1, DH), lambda m, k: (0, 0)),
        ],
        out_specs=pl.BlockSpec((m_blk, DH), lambda m, k: (m, 0)),
        out_shape=jax.ShapeDtypeStruct((N, DH), jnp.float32),
        scratch_shapes=[pltpu.VMEM((m_blk, DH), jnp.float32)],
        compiler_params=pltpu.CompilerParams(
            dimension_semantics=("parallel", "arbitrary")),
    )


def _chunked(h, num_chunks):
    return h.reshape(N, num_chunks, LANE).transpose(1, 0, 2)


def _wbig(w_in, w_out, w_lin, num_chunks):
    wcat = jnp.concatenate(
        [0.5 * w_in, 0.5 * w_out, jnp.sum(w_lin, axis=0)[None]], axis=0)
    d = num_chunks * LANE
    return wcat.reshape(9, num_chunks, LANE, DH).reshape(9 * num_chunks, LANE, DH)


def kernel(x, edge_index_forward, edge_index_onset, edge_index_sustain,
           edge_index_rest, W1_in, b1_in, W1_out, b1_out, W1_lin, b1_lin,
           W2_in, b2_in, W2_out, b2_out, W2_lin, b2_lin,
           ln1_g, ln1_b, ln2_g, ln2_b):
    eis = (edge_index_forward, edge_index_onset, edge_index_sustain,
           edge_index_rest)
    gidx = jnp.stack([e[0] for e in eis] + [e[1] for e in eis])
    sidx = jnp.stack([e[1] for e in eis] + [e[0] for e in eis])
    gidx = gidx.reshape(8, NT, NB, B)
    sidx = sidx.reshape(8, NT, NB, B)
    zrows = jnp.zeros((RPT_LAST, LANE), jnp.float32)

    def layer(h, num_chunks, w_in, b_in, w_out, b_out, w_lin, b_lin, g, b):
        h_r = _chunked(h, num_chunks)
        agg = _sc_aggregate(num_chunks)(h_r, gidx, sidx, zrows)
        wb = _wbig(w_in, w_out, w_lin, num_chunks)
        btot = jnp.sum(0.5 * b_in + 0.5 * b_out + b_lin, axis=0)[None]
        return _tc_matmul_ln(num_chunks)(
            agg, h_r, wb, btot, g[None], b[None])

    h1 = layer(x, 2, W1_in, b1_in, W1_out, b1_out, W1_lin, b1_lin,
               ln1_g, ln1_b)
    h2 = layer(h1, 4, W2_in, b2_in, W2_out, b2_out, W2_lin, b2_lin,
               ln2_g, ln2_b)
    return h2


# speedup vs baseline: 6.2740x; 1.5599x over previous
import functools

import jax
import jax.numpy as jnp
from jax import lax
from jax.experimental import pallas as pl
from jax.experimental.pallas import tpu as pltpu
from jax.experimental.pallas import tpu_sc as plsc

N = 10000
E = 160000
DH = 512
LANE = 128
NT = 16
NCORE = 2
EPT = E // NT
B = 100
NB = EPT // B
NBUF = 2
NH = 2
HNB = NB // NH
RPT = 624
RPT_LAST = N - (NT - 1) * RPT


def _sc_aggregate(num_chunks):
    cpc = num_chunks // NCORE

    def body(x_hbm, g_hbm, s_hbm, z_hbm, out_hbm,
             accum, gbuf, sbuf, rows, *sems):
        cid = lax.axis_index("c")
        sid = lax.axis_index("s")
        row0 = sid * RPT

        last = sid == NT - 1
        for cc in range(cpc):
            chunk = cid * cpc + cc
            for j in range(8):
                @pl.when(jnp.logical_not(last))
                def _():
                    pltpu.sync_copy(z_hbm.at[pl.ds(0, RPT)],
                                    accum.at[pl.ds(row0, RPT)])

                @pl.when(last)
                def _():
                    pltpu.sync_copy(z_hbm, accum.at[pl.ds(row0, RPT_LAST)])

                plsc.subcore_barrier()

                for h in range(NH):
                    pltpu.sync_copy(g_hbm.at[j, sid * NH + h], gbuf)
                    pltpu.sync_copy(s_hbm.at[j, sid * NH + h], sbuf)
                    for k in range(NBUF):
                        pltpu.async_copy(
                            x_hbm.at[chunk].at[gbuf.at[k]], rows.at[k],
                            sems[k])

                    @pl.loop(0, HNB, step=NBUF)
                    def _(g):
                        for k in range(NBUF):
                            b = g + k
                            pltpu.make_async_copy(
                                x_hbm.at[chunk].at[gbuf.at[b]], rows.at[k],
                                sems[k]).wait()
                            pltpu.sync_copy(
                                rows.at[k], accum.at[sbuf.at[b]], add=True)
                            nxt = b + NBUF

                            @pl.when(nxt < HNB)
                            def _():
                                pltpu.async_copy(
                                    x_hbm.at[chunk].at[gbuf.at[nxt]],
                                    rows.at[k], sems[k])

                plsc.subcore_barrier()

                @pl.when(jnp.logical_not(last))
                def _():
                    pltpu.sync_copy(
                        accum.at[pl.ds(row0, RPT)],
                        out_hbm.at[j, chunk, pl.ds(row0, RPT)])

                @pl.when(last)
                def _():
                    pltpu.sync_copy(
                        accum.at[pl.ds(row0, RPT_LAST)],
                        out_hbm.at[j, chunk, pl.ds(row0, RPT_LAST)])

    mesh = plsc.VectorSubcoreMesh(core_axis_name="c", subcore_axis_name="s")
    return pl.kernel(
        body,
        out_type=jax.ShapeDtypeStruct((8, num_chunks, N, LANE), jnp.float32),
        mesh=mesh,
        scratch_types=[
            pltpu.VMEM_SHARED((N, LANE), jnp.float32),
            pltpu.VMEM((HNB, B), jnp.int32),
            pltpu.VMEM((HNB, B), jnp.int32),
            pltpu.VMEM((NBUF, B, LANE), jnp.float32),
        ] + [pltpu.SemaphoreType.DMA] * NBUF,
    )


def _tc_matmul_ln(num_chunks, m_blk=2000):
    K = 9 * num_chunks
    grid = (N // m_blk, K)

    def body(agg_ref, x_ref, w_ref, bt_ref, g_ref, b_ref, out_ref, acc_ref):
        k = pl.program_id(1)

        @pl.when(k == 0)
        def _():
            acc_ref[...] = jnp.zeros_like(acc_ref)

        piece = jnp.where(k < 8 * num_chunks, agg_ref[0, 0], x_ref[0])
        acc_ref[...] += jax.lax.dot(
            piece, w_ref[0], preferred_element_type=jnp.float32)

        @pl.when(k == K - 1)
        def _():
            y = acc_ref[...] + bt_ref[...]
            mu = jnp.mean(y, axis=-1, keepdims=True)
            var = jnp.mean((y - mu) ** 2, axis=-1, keepdims=True)
            out_ref[...] = ((y - mu) * jax.lax.rsqrt(var + 1e-5)
                            * g_ref[...] + b_ref[...])

    return pl.pallas_call(
        body,
        grid=grid,
        in_specs=[
            pl.BlockSpec((1, 1, m_blk, LANE),
                         lambda m, k: (jnp.minimum(k // num_chunks, 7),
                                       k % num_chunks, m, 0)),
            pl.BlockSpec((1, m_blk, LANE), lambda m, k: (k % num_chunks, m, 0)),
            pl.BlockSpec((1, LANE, DH), lambda m, k: (k, 0, 0)),
            pl.BlockSpec((1, DH), lambda m, k: (0, 0)),
            pl.BlockSpec((1, DH), lambda m, k: (0, 0)),
            pl.BlockSpec((1, DH), lambda m, k: (0, 0)),
        ],
        out_specs=pl.BlockSpec((m_blk, DH), lambda m, k: (m, 0)),
        out_shape=jax.ShapeDtypeStruct((N, DH), jnp.float32),
        scratch_shapes=[pltpu.VMEM((m_blk, DH), jnp.float32)],
        compiler_params=pltpu.CompilerParams(
            dimension_semantics=("parallel", "arbitrary")),
    )


def _chunked(h, num_chunks):
    return h.reshape(N, num_chunks, LANE).transpose(1, 0, 2)


def _wbig(w_in, w_out, w_lin, num_chunks):
    wcat = jnp.concatenate(
        [0.5 * w_in, 0.5 * w_out, jnp.sum(w_lin, axis=0)[None]], axis=0)
    d = num_chunks * LANE
    return wcat.reshape(9, num_chunks, LANE, DH).reshape(9 * num_chunks, LANE, DH)


def kernel(x, edge_index_forward, edge_index_onset, edge_index_sustain,
           edge_index_rest, W1_in, b1_in, W1_out, b1_out, W1_lin, b1_lin,
           W2_in, b2_in, W2_out, b2_out, W2_lin, b2_lin,
           ln1_g, ln1_b, ln2_g, ln2_b):
    eis = (edge_index_forward, edge_index_onset, edge_index_sustain,
           edge_index_rest)
    gidx = jnp.stack([e[0] for e in eis] + [e[1] for e in eis])
    sidx = jnp.stack([e[1] for e in eis] + [e[0] for e in eis])
    gidx = gidx.reshape(8, NT * NH, HNB, B)
    sidx = sidx.reshape(8, NT * NH, HNB, B)
    zrows = jnp.zeros((RPT_LAST, LANE), jnp.float32)

    def layer(h, num_chunks, w_in, b_in, w_out, b_out, w_lin, b_lin, g, b):
        h_r = _chunked(h, num_chunks)
        agg = _sc_aggregate(num_chunks)(h_r, gidx, sidx, zrows)
        wb = _wbig(w_in, w_out, w_lin, num_chunks)
        btot = jnp.sum(0.5 * b_in + 0.5 * b_out + b_lin, axis=0)[None]
        return _tc_matmul_ln(num_chunks)(
            agg, h_r, wb, btot, g[None], b[None])

    h1 = layer(x, 2, W1_in, b1_in, W1_out, b1_out, W1_lin, b1_lin,
               ln1_g, ln1_b)
    h2 = layer(h1, 4, W2_in, b2_in, W2_out, b2_out, W2_lin, b2_lin,
               ln2_g, ln2_b)
    return h2


# speedup vs baseline: 6.5553x; 1.0448x over previous
import functools

import jax
import jax.numpy as jnp
from jax import lax
from jax.experimental import pallas as pl
from jax.experimental.pallas import tpu as pltpu
from jax.experimental.pallas import tpu_sc as plsc

N = 10000
E = 160000
DH = 512
LANE = 128
NT = 16
NCORE = 2
EPT = E // NT
B = 125
NB = EPT // B
NBUF = 2
NH = 2
HNB = NB // NH
RPT = 624
RPT_LAST = N - (NT - 1) * RPT


def _sc_aggregate(num_chunks):
    cpc = num_chunks // NCORE

    def body(x_hbm, g_hbm, s_hbm, z_hbm, out_hbm,
             accum, gbuf, sbuf, rows, *sems):
        cid = lax.axis_index("c")
        sid = lax.axis_index("s")
        row0 = sid * RPT

        last = sid == NT - 1
        for cc in range(cpc):
            chunk = cid * cpc + cc
            for j in range(8):
                @pl.when(jnp.logical_not(last))
                def _():
                    pltpu.sync_copy(z_hbm.at[pl.ds(0, RPT)],
                                    accum.at[pl.ds(row0, RPT)])

                @pl.when(last)
                def _():
                    pltpu.sync_copy(z_hbm, accum.at[pl.ds(row0, RPT_LAST)])

                plsc.subcore_barrier()

                gsems = sems[:NBUF]
                ssems = sems[NBUF:]
                for h in range(NH):
                    pltpu.sync_copy(g_hbm.at[j, sid * NH + h], gbuf)
                    pltpu.sync_copy(s_hbm.at[j, sid * NH + h], sbuf)
                    for k in range(NBUF):
                        pltpu.async_copy(
                            x_hbm.at[chunk].at[gbuf.at[k]], rows.at[k],
                            gsems[k])

                    @pl.loop(0, HNB, step=NBUF)
                    def _(g):
                        for k in range(NBUF):
                            b = g + k
                            pltpu.make_async_copy(
                                x_hbm.at[chunk].at[gbuf.at[b]], rows.at[k],
                                gsems[k]).wait()
                            pltpu.async_copy(
                                rows.at[k], accum.at[sbuf.at[b]], ssems[k],
                                add=True)
                            nxt = b + NBUF

                            @pl.when(nxt < HNB)
                            def _():
                                pltpu.make_async_copy(
                                    rows.at[k], accum.at[sbuf.at[b]],
                                    ssems[k]).wait()
                                pltpu.async_copy(
                                    x_hbm.at[chunk].at[gbuf.at[nxt]],
                                    rows.at[k], gsems[k])

                    for k in range(NBUF):
                        pltpu.make_async_copy(
                            rows.at[k], accum.at[sbuf.at[HNB - NBUF + k]],
                            ssems[k]).wait()

                plsc.subcore_barrier()

                @pl.when(jnp.logical_not(last))
                def _():
                    pltpu.sync_copy(
                        accum.at[pl.ds(row0, RPT)],
                        out_hbm.at[j, chunk, pl.ds(row0, RPT)])

                @pl.when(last)
                def _():
                    pltpu.sync_copy(
                        accum.at[pl.ds(row0, RPT_LAST)],
                        out_hbm.at[j, chunk, pl.ds(row0, RPT_LAST)])

    mesh = plsc.VectorSubcoreMesh(core_axis_name="c", subcore_axis_name="s")
    return pl.kernel(
        body,
        out_type=jax.ShapeDtypeStruct((8, num_chunks, N, LANE), jnp.float32),
        mesh=mesh,
        scratch_types=[
            pltpu.VMEM_SHARED((N, LANE), jnp.float32),
            pltpu.VMEM((HNB, B), jnp.int32),
            pltpu.VMEM((HNB, B), jnp.int32),
            pltpu.VMEM((NBUF, B, LANE), jnp.float32),
        ] + [pltpu.SemaphoreType.DMA] * (2 * NBUF),
    )


def _tc_matmul_ln(num_chunks, m_blk=2000):
    K = 9 * num_chunks
    grid = (N // m_blk, K)

    def body(agg_ref, x_ref, w_ref, bt_ref, g_ref, b_ref, out_ref, acc_ref):
        k = pl.program_id(1)

        @pl.when(k == 0)
        def _():
            acc_ref[...] = jnp.zeros_like(acc_ref)

        piece = jnp.where(k < 8 * num_chunks, agg_ref[0, 0], x_ref[0])
        acc_ref[...] += jax.lax.dot(
            piece, w_ref[0], preferred_element_type=jnp.float32)

        @pl.when(k == K - 1)
        def _():
            y = acc_ref[...] + bt_ref[...]
            mu = jnp.mean(y, axis=-1, keepdims=True)
            var = jnp.mean((y - mu) ** 2, axis=-1, keepdims=True)
            out_ref[...] = ((y - mu) * jax.lax.rsqrt(var + 1e-5)
                            * g_ref[...] + b_ref[...])

    return pl.pallas_call(
        body,
        grid=grid,
        in_specs=[
            pl.BlockSpec((1, 1, m_blk, LANE),
                         lambda m, k: (jnp.minimum(k // num_chunks, 7),
                                       k % num_chunks, m, 0)),
            pl.BlockSpec((1, m_blk, LANE), lambda m, k: (k % num_chunks, m, 0)),
            pl.BlockSpec((1, LANE, DH), lambda m, k: (k, 0, 0)),
            pl.BlockSpec((1, DH), lambda m, k: (0, 0)),
            pl.BlockSpec((1, DH), lambda m, k: (0, 0)),
            pl.BlockSpec((1, DH), lambda m, k: (0, 0)),
        ],
        out_specs=pl.BlockSpec((m_blk, DH), lambda m, k: (m, 0)),
        out_shape=jax.ShapeDtypeStruct((N, DH), jnp.float32),
        scratch_shapes=[pltpu.VMEM((m_blk, DH), jnp.float32)],
        compiler_params=pltpu.CompilerParams(
            dimension_semantics=("parallel", "arbitrary")),
    )


def _chunked(h, num_chunks):
    return h.reshape(N, num_chunks, LANE).transpose(1, 0, 2)


def _wbig(w_in, w_out, w_lin, num_chunks):
    wcat = jnp.concatenate(
        [0.5 * w_in, 0.5 * w_out, jnp.sum(w_lin, axis=0)[None]], axis=0)
    d = num_chunks * LANE
    return wcat.reshape(9, num_chunks, LANE, DH).reshape(9 * num_chunks, LANE, DH)


def kernel(x, edge_index_forward, edge_index_onset, edge_index_sustain,
           edge_index_rest, W1_in, b1_in, W1_out, b1_out, W1_lin, b1_lin,
           W2_in, b2_in, W2_out, b2_out, W2_lin, b2_lin,
           ln1_g, ln1_b, ln2_g, ln2_b):
    eis = (edge_index_forward, edge_index_onset, edge_index_sustain,
           edge_index_rest)
    gidx = jnp.stack([e[0] for e in eis] + [e[1] for e in eis])
    sidx = jnp.stack([e[1] for e in eis] + [e[0] for e in eis])
    gidx = gidx.reshape(8, NT * NH, HNB, B)
    sidx = sidx.reshape(8, NT * NH, HNB, B)
    zrows = jnp.zeros((RPT_LAST, LANE), jnp.float32)

    def layer(h, num_chunks, w_in, b_in, w_out, b_out, w_lin, b_lin, g, b):
        h_r = _chunked(h, num_chunks)
        agg = _sc_aggregate(num_chunks)(h_r, gidx, sidx, zrows)
        wb = _wbig(w_in, w_out, w_lin, num_chunks)
        btot = jnp.sum(0.5 * b_in + 0.5 * b_out + b_lin, axis=0)[None]
        return _tc_matmul_ln(num_chunks)(
            agg, h_r, wb, btot, g[None], b[None])

    h1 = layer(x, 2, W1_in, b1_in, W1_out, b1_out, W1_lin, b1_lin,
               ln1_g, ln1_b)
    h2 = layer(h1, 4, W2_in, b2_in, W2_out, b2_out, W2_lin, b2_lin,
               ln2_g, ln2_b)
    return h2


# speedup vs baseline: 6.5903x; 1.0053x over previous
import functools

import jax
import jax.numpy as jnp
from jax import lax
from jax.experimental import pallas as pl
from jax.experimental.pallas import tpu as pltpu
from jax.experimental.pallas import tpu_sc as plsc

N = 10000
E = 160000
DH = 512
LANE = 128
NT = 16
NCORE = 2
EPT = E // NT
B = 125
NB = EPT // B
NBUF = 2
NH = 2
HNB = NB // NH
RPT = 624
RPT_LAST = N - (NT - 1) * RPT


def _sc_aggregate(num_chunks):
    cpc = num_chunks // NCORE

    def body(x_hbm, g_hbm, s_hbm, z_hbm, out_hbm,
             accum, gbuf, sbuf, rows, *sems):
        cid = lax.axis_index("c")
        sid = lax.axis_index("s")
        row0 = sid * RPT

        last = sid == NT - 1
        for cc in range(cpc):
            chunk = cid * cpc + cc
            for j in range(8):
                @pl.when(jnp.logical_not(last))
                def _():
                    pltpu.sync_copy(z_hbm.at[pl.ds(0, RPT)],
                                    accum.at[pl.ds(row0, RPT)])

                @pl.when(last)
                def _():
                    pltpu.sync_copy(z_hbm, accum.at[pl.ds(row0, RPT_LAST)])

                plsc.subcore_barrier()

                gsems = sems[:NBUF]
                ssems = sems[NBUF:]
                for h in range(NH):
                    pltpu.sync_copy(g_hbm.at[j, sid * NH + h], gbuf)
                    pltpu.sync_copy(s_hbm.at[j, sid * NH + h], sbuf)
                    for k in range(NBUF):
                        pltpu.async_copy(
                            x_hbm.at[chunk].at[gbuf.at[k]], rows.at[k],
                            gsems[k])

                    @pl.loop(0, HNB, step=NBUF)
                    def _(g):
                        for k in range(NBUF):
                            b = g + k
                            pltpu.make_async_copy(
                                x_hbm.at[chunk].at[gbuf.at[b]], rows.at[k],
                                gsems[k]).wait()
                            pltpu.async_copy(
                                rows.at[k], accum.at[sbuf.at[b]], ssems[k],
                                add=True)
                            nxt = b + NBUF

                            @pl.when(nxt < HNB)
                            def _():
                                pltpu.make_async_copy(
                                    rows.at[k], accum.at[sbuf.at[b]],
                                    ssems[k]).wait()
                                pltpu.async_copy(
                                    x_hbm.at[chunk].at[gbuf.at[nxt]],
                                    rows.at[k], gsems[k])

                    for k in range(NBUF):
                        pltpu.make_async_copy(
                            rows.at[k], accum.at[sbuf.at[HNB - NBUF + k]],
                            ssems[k]).wait()

                plsc.subcore_barrier()

                @pl.when(jnp.logical_not(last))
                def _():
                    pltpu.sync_copy(
                        accum.at[pl.ds(row0, RPT)],
                        out_hbm.at[j, chunk, pl.ds(row0, RPT)])

                @pl.when(last)
                def _():
                    pltpu.sync_copy(
                        accum.at[pl.ds(row0, RPT_LAST)],
                        out_hbm.at[j, chunk, pl.ds(row0, RPT_LAST)])

    mesh = plsc.VectorSubcoreMesh(core_axis_name="c", subcore_axis_name="s")
    return pl.kernel(
        body,
        out_type=jax.ShapeDtypeStruct((8, num_chunks, N, LANE), jnp.float32),
        mesh=mesh,
        scratch_types=[
            pltpu.VMEM_SHARED((N, LANE), jnp.float32),
            pltpu.VMEM((HNB, B), jnp.int32),
            pltpu.VMEM((HNB, B), jnp.int32),
            pltpu.VMEM((NBUF, B, LANE), jnp.float32),
        ] + [pltpu.SemaphoreType.DMA] * (2 * NBUF),
    )


def _tc_matmul_ln(num_chunks, m_blk=2000):
    K = 9 * num_chunks
    grid = (N // m_blk, K)

    def body(agg_ref, x_ref, w_ref, bt_ref, g_ref, b_ref, out_ref, acc_ref):
        k = pl.program_id(1)

        @pl.when(k == 0)
        def _():
            acc_ref[...] = jnp.zeros_like(acc_ref)

        piece = jnp.where(k < 8 * num_chunks, agg_ref[0, 0], x_ref[0])
        acc_ref[...] += jax.lax.dot(
            piece.astype(jnp.bfloat16), w_ref[0],
            preferred_element_type=jnp.float32)

        @pl.when(k == K - 1)
        def _():
            y = acc_ref[...] + bt_ref[...]
            mu = jnp.mean(y, axis=-1, keepdims=True)
            var = jnp.mean((y - mu) ** 2, axis=-1, keepdims=True)
            out_ref[...] = ((y - mu) * jax.lax.rsqrt(var + 1e-5)
                            * g_ref[...] + b_ref[...])

    return pl.pallas_call(
        body,
        grid=grid,
        in_specs=[
            pl.BlockSpec((1, 1, m_blk, LANE),
                         lambda m, k: (jnp.minimum(k // num_chunks, 7),
                                       k % num_chunks, m, 0)),
            pl.BlockSpec((1, m_blk, LANE), lambda m, k: (k % num_chunks, m, 0)),
            pl.BlockSpec((1, LANE, DH), lambda m, k: (k, 0, 0)),
            pl.BlockSpec((1, DH), lambda m, k: (0, 0)),
            pl.BlockSpec((1, DH), lambda m, k: (0, 0)),
            pl.BlockSpec((1, DH), lambda m, k: (0, 0)),
        ],
        out_specs=pl.BlockSpec((m_blk, DH), lambda m, k: (m, 0)),
        out_shape=jax.ShapeDtypeStruct((N, DH), jnp.float32),
        scratch_shapes=[pltpu.VMEM((m_blk, DH), jnp.float32)],
        compiler_params=pltpu.CompilerParams(
            dimension_semantics=("parallel", "arbitrary")),
    )


def _chunked(h, num_chunks):
    return h.reshape(N, num_chunks, LANE).transpose(1, 0, 2)


def _wbig(w_in, w_out, w_lin, num_chunks):
    wcat = jnp.concatenate(
        [0.5 * w_in, 0.5 * w_out, jnp.sum(w_lin, axis=0)[None]], axis=0)
    d = num_chunks * LANE
    return wcat.reshape(9, num_chunks, LANE, DH).reshape(9 * num_chunks, LANE, DH)


def kernel(x, edge_index_forward, edge_index_onset, edge_index_sustain,
           edge_index_rest, W1_in, b1_in, W1_out, b1_out, W1_lin, b1_lin,
           W2_in, b2_in, W2_out, b2_out, W2_lin, b2_lin,
           ln1_g, ln1_b, ln2_g, ln2_b):
    eis = (edge_index_forward, edge_index_onset, edge_index_sustain,
           edge_index_rest)
    gidx = jnp.stack([e[0] for e in eis] + [e[1] for e in eis])
    sidx = jnp.stack([e[1] for e in eis] + [e[0] for e in eis])
    gidx = gidx.reshape(8, NT * NH, HNB, B)
    sidx = sidx.reshape(8, NT * NH, HNB, B)
    zrows = jnp.zeros((RPT_LAST, LANE), jnp.float32)

    def layer(h, num_chunks, w_in, b_in, w_out, b_out, w_lin, b_lin, g, b):
        h_r = _chunked(h, num_chunks)
        agg = _sc_aggregate(num_chunks)(h_r, gidx, sidx, zrows)
        wb = _wbig(w_in, w_out, w_lin, num_chunks).astype(jnp.bfloat16)
        btot = jnp.sum(0.5 * b_in + 0.5 * b_out + b_lin, axis=0)[None]
        return _tc_matmul_ln(num_chunks)(
            agg, h_r, wb, btot, g[None], b[None])

    h1 = layer(x, 2, W1_in, b1_in, W1_out, b1_out, W1_lin, b1_lin,
               ln1_g, ln1_b)
    h2 = layer(h1, 4, W2_in, b2_in, W2_out, b2_out, W2_lin, b2_lin,
               ln2_g, ln2_b)
    return h2


# speedup vs baseline: 6.9992x; 1.0621x over previous
import functools

import jax
import jax.numpy as jnp
from jax import lax
from jax.experimental import pallas as pl
from jax.experimental.pallas import tpu as pltpu
from jax.experimental.pallas import tpu_sc as plsc

N = 10000
E = 160000
DH = 512
LANE = 128
NT = 16
NCORE = 2
EPT = E // NT
B = 125
NB = EPT // B
NBUF = 2
NH = 2
HNB = NB // NH
RPT = 624
RPT_LAST = N - (NT - 1) * RPT


def _sc_aggregate(num_chunks, j0, nj):
    cpc = num_chunks // NCORE

    def body(x_hbm, g_hbm, s_hbm, z_hbm, out_hbm,
             accum, gbuf, sbuf, rows, *sems):
        cid = lax.axis_index("c")
        sid = lax.axis_index("s")
        row0 = sid * RPT

        last = sid == NT - 1
        for cc in range(cpc):
            chunk = cid * cpc + cc
            for jj in range(nj):
                j = j0 + jj
                @pl.when(jnp.logical_not(last))
                def _():
                    pltpu.sync_copy(z_hbm.at[pl.ds(0, RPT)],
                                    accum.at[pl.ds(row0, RPT)])

                @pl.when(last)
                def _():
                    pltpu.sync_copy(z_hbm, accum.at[pl.ds(row0, RPT_LAST)])

                plsc.subcore_barrier()

                gsems = sems[:NBUF]
                ssems = sems[NBUF:]
                for h in range(NH):
                    pltpu.sync_copy(g_hbm.at[j, sid * NH + h], gbuf)
                    pltpu.sync_copy(s_hbm.at[j, sid * NH + h], sbuf)
                    for k in range(NBUF):
                        pltpu.async_copy(
                            x_hbm.at[chunk].at[gbuf.at[k]], rows.at[k],
                            gsems[k])

                    @pl.loop(0, HNB, step=NBUF)
                    def _(g):
                        for k in range(NBUF):
                            b = g + k
                            pltpu.make_async_copy(
                                x_hbm.at[chunk].at[gbuf.at[b]], rows.at[k],
                                gsems[k]).wait()
                            pltpu.async_copy(
                                rows.at[k], accum.at[sbuf.at[b]], ssems[k],
                                add=True)
                            nxt = b + NBUF

                            @pl.when(nxt < HNB)
                            def _():
                                pltpu.make_async_copy(
                                    rows.at[k], accum.at[sbuf.at[b]],
                                    ssems[k]).wait()
                                pltpu.async_copy(
                                    x_hbm.at[chunk].at[gbuf.at[nxt]],
                                    rows.at[k], gsems[k])

                    for k in range(NBUF):
                        pltpu.make_async_copy(
                            rows.at[k], accum.at[sbuf.at[HNB - NBUF + k]],
                            ssems[k]).wait()

                plsc.subcore_barrier()

                @pl.when(jnp.logical_not(last))
                def _():
                    pltpu.sync_copy(
                        accum.at[pl.ds(row0, RPT)],
                        out_hbm.at[jj, chunk, pl.ds(row0, RPT)])

                @pl.when(last)
                def _():
                    pltpu.sync_copy(
                        accum.at[pl.ds(row0, RPT_LAST)],
                        out_hbm.at[jj, chunk, pl.ds(row0, RPT_LAST)])

    mesh = plsc.VectorSubcoreMesh(core_axis_name="c", subcore_axis_name="s")
    return pl.kernel(
        body,
        out_type=jax.ShapeDtypeStruct((nj, num_chunks, N, LANE), jnp.float32),
        mesh=mesh,
        scratch_types=[
            pltpu.VMEM_SHARED((N, LANE), jnp.float32),
            pltpu.VMEM((HNB, B), jnp.int32),
            pltpu.VMEM((HNB, B), jnp.int32),
            pltpu.VMEM((NBUF, B, LANE), jnp.float32),
        ] + [pltpu.SemaphoreType.DMA] * (2 * NBUF),
    )


def _tc_partial(num_chunks, nj, m_blk=2000):
    K = (nj + 1) * num_chunks
    grid = (N // m_blk, K)

    def body(agg_ref, x_ref, w_ref, out_ref, acc_ref):
        k = pl.program_id(1)

        @pl.when(k == 0)
        def _():
            acc_ref[...] = jnp.zeros_like(acc_ref)

        piece = jnp.where(k < nj * num_chunks, agg_ref[0, 0], x_ref[0])
        acc_ref[...] += jax.lax.dot(
            piece.astype(jnp.bfloat16), w_ref[0],
            preferred_element_type=jnp.float32)

        @pl.when(k == K - 1)
        def _():
            out_ref[...] = acc_ref[...]

    return pl.pallas_call(
        body,
        grid=grid,
        in_specs=[
            pl.BlockSpec((1, 1, m_blk, LANE),
                         lambda m, k: (jnp.minimum(k // num_chunks, nj - 1),
                                       k % num_chunks, m, 0)),
            pl.BlockSpec((1, m_blk, LANE), lambda m, k: (k % num_chunks, m, 0)),
            pl.BlockSpec((1, LANE, DH), lambda m, k: (k, 0, 0)),
        ],
        out_specs=pl.BlockSpec((m_blk, DH), lambda m, k: (m, 0)),
        out_shape=jax.ShapeDtypeStruct((N, DH), jnp.float32),
        scratch_shapes=[pltpu.VMEM((m_blk, DH), jnp.float32)],
        compiler_params=pltpu.CompilerParams(
            dimension_semantics=("parallel", "arbitrary")),
    )


def _tc_final(num_chunks, nj, m_blk=2000):
    K = nj * num_chunks
    grid = (N // m_blk, K)

    def body(agg_ref, p_ref, w_ref, bt_ref, g_ref, b_ref, out_ref, acc_ref):
        k = pl.program_id(1)

        @pl.when(k == 0)
        def _():
            acc_ref[...] = p_ref[...]

        acc_ref[...] += jax.lax.dot(
            agg_ref[0, 0].astype(jnp.bfloat16), w_ref[0],
            preferred_element_type=jnp.float32)

        @pl.when(k == K - 1)
        def _():
            y = acc_ref[...] + bt_ref[...]
            mu = jnp.mean(y, axis=-1, keepdims=True)
            var = jnp.mean((y - mu) ** 2, axis=-1, keepdims=True)
            out_ref[...] = ((y - mu) * jax.lax.rsqrt(var + 1e-5)
                            * g_ref[...] + b_ref[...])

    return pl.pallas_call(
        body,
        grid=grid,
        in_specs=[
            pl.BlockSpec((1, 1, m_blk, LANE),
                         lambda m, k: (k // num_chunks, k % num_chunks, m, 0)),
            pl.BlockSpec((m_blk, DH), lambda m, k: (m, 0)),
            pl.BlockSpec((1, LANE, DH), lambda m, k: (k, 0, 0)),
            pl.BlockSpec((1, DH), lambda m, k: (0, 0)),
            pl.BlockSpec((1, DH), lambda m, k: (0, 0)),
            pl.BlockSpec((1, DH), lambda m, k: (0, 0)),
        ],
        out_specs=pl.BlockSpec((m_blk, DH), lambda m, k: (m, 0)),
        out_shape=jax.ShapeDtypeStruct((N, DH), jnp.float32),
        scratch_shapes=[pltpu.VMEM((m_blk, DH), jnp.float32)],
        compiler_params=pltpu.CompilerParams(
            dimension_semantics=("parallel", "arbitrary")),
    )


def _chunked(h, num_chunks):
    return h.reshape(N, num_chunks, LANE).transpose(1, 0, 2)


def _wbig(w_in, w_out, w_lin, num_chunks):
    wcat = jnp.concatenate(
        [0.5 * w_in, 0.5 * w_out, jnp.sum(w_lin, axis=0)[None]], axis=0)
    d = num_chunks * LANE
    return wcat.reshape(9, num_chunks, LANE, DH).reshape(9 * num_chunks, LANE, DH)


def kernel(x, edge_index_forward, edge_index_onset, edge_index_sustain,
           edge_index_rest, W1_in, b1_in, W1_out, b1_out, W1_lin, b1_lin,
           W2_in, b2_in, W2_out, b2_out, W2_lin, b2_lin,
           ln1_g, ln1_b, ln2_g, ln2_b):
    eis = (edge_index_forward, edge_index_onset, edge_index_sustain,
           edge_index_rest)
    gidx = jnp.stack([e[0] for e in eis] + [e[1] for e in eis])
    sidx = jnp.stack([e[1] for e in eis] + [e[0] for e in eis])
    gidx = gidx.reshape(8, NT * NH, HNB, B)
    sidx = sidx.reshape(8, NT * NH, HNB, B)
    zrows = jnp.zeros((RPT_LAST, LANE), jnp.float32)

    NJA = 6

    def layer(h, num_chunks, w_in, b_in, w_out, b_out, w_lin, b_lin, g, b):
        h_r = _chunked(h, num_chunks)
        agg_a = _sc_aggregate(num_chunks, 0, NJA)(h_r, gidx, sidx, zrows)
        agg_b = _sc_aggregate(num_chunks, NJA, 8 - NJA)(h_r, gidx, sidx, zrows)
        w9 = _wbig(w_in, w_out, w_lin, num_chunks)
        wa = jnp.concatenate(
            [w9[:NJA * num_chunks], w9[8 * num_chunks:]],
            axis=0).astype(jnp.bfloat16)
        wf = w9[NJA * num_chunks:8 * num_chunks].astype(jnp.bfloat16)
        btot = jnp.sum(0.5 * b_in + 0.5 * b_out + b_lin, axis=0)[None]
        part = _tc_partial(num_chunks, NJA)(agg_a, h_r, wa)
        return _tc_final(num_chunks, 8 - NJA)(
            agg_b, part, wf, btot, g[None], b[None])

    h1 = layer(x, 2, W1_in, b1_in, W1_out, b1_out, W1_lin, b1_lin,
               ln1_g, ln1_b)
    h2 = layer(h1, 4, W2_in, b2_in, W2_out, b2_out, W2_lin, b2_lin,
               ln2_g, ln2_b)
    return h2


# speedup vs baseline: 7.1607x; 1.0231x over previous
import functools

import jax
import jax.numpy as jnp
from jax import lax
from jax.experimental import pallas as pl
from jax.experimental.pallas import tpu as pltpu
from jax.experimental.pallas import tpu_sc as plsc

N = 10000
E = 160000
DH = 512
LANE = 128
NT = 16
NCORE = 2
EPT = E // NT
B = 125
NB = EPT // B
NBUF = 2
NH = 2
HNB = NB // NH
RPT = 624
RPT_LAST = N - (NT - 1) * RPT


def _sc_aggregate(num_chunks, j0, nj):
    cpc = num_chunks // NCORE

    def body(x_hbm, g_hbm, s_hbm, z_hbm, out_hbm,
             accum, gbuf, sbuf, rows, *sems):
        cid = lax.axis_index("c")
        sid = lax.axis_index("s")
        row0 = sid * RPT

        last = sid == NT - 1
        gsems = sems[:NBUF]
        ssems = sems[NBUF:2 * NBUF]
        wbsem = sems[2 * NBUF]

        def wb_copy(jj, chunk):
            return (
                pltpu.make_async_copy(
                    accum.at[pl.ds(row0, RPT)],
                    out_hbm.at[jj, chunk, pl.ds(row0, RPT)], wbsem),
                pltpu.make_async_copy(
                    accum.at[pl.ds(row0, RPT_LAST)],
                    out_hbm.at[jj, chunk, pl.ds(row0, RPT_LAST)], wbsem),
            )

        tasks = [(cc, jj) for cc in range(cpc) for jj in range(nj)]
        for ti, (cc, jj) in enumerate(tasks):
            chunk = cid * cpc + cc
            j = j0 + jj

            pltpu.sync_copy(g_hbm.at[j, sid * NH], gbuf)
            pltpu.sync_copy(s_hbm.at[j, sid * NH], sbuf)
            for k in range(NBUF):
                pltpu.async_copy(
                    x_hbm.at[chunk].at[gbuf.at[k]], rows.at[k], gsems[k])

            if ti > 0:
                pcc, pjj = tasks[ti - 1]
                wnorm, wlast = wb_copy(pjj, cid * cpc + pcc)

                @pl.when(jnp.logical_not(last))
                def _():
                    wnorm.wait()

                @pl.when(last)
                def _():
                    wlast.wait()

            @pl.when(jnp.logical_not(last))
            def _():
                pltpu.sync_copy(z_hbm.at[pl.ds(0, RPT)],
                                accum.at[pl.ds(row0, RPT)])

            @pl.when(last)
            def _():
                pltpu.sync_copy(z_hbm, accum.at[pl.ds(row0, RPT_LAST)])

            plsc.subcore_barrier()

            for h in range(NH):
                if h > 0:
                    pltpu.sync_copy(g_hbm.at[j, sid * NH + h], gbuf)
                    pltpu.sync_copy(s_hbm.at[j, sid * NH + h], sbuf)
                    for k in range(NBUF):
                        pltpu.async_copy(
                            x_hbm.at[chunk].at[gbuf.at[k]], rows.at[k],
                            gsems[k])

                @pl.loop(0, HNB, step=NBUF)
                def _(g):
                    for k in range(NBUF):
                        b = g + k
                        pltpu.make_async_copy(
                            x_hbm.at[chunk].at[gbuf.at[b]], rows.at[k],
                            gsems[k]).wait()
                        pltpu.async_copy(
                            rows.at[k], accum.at[sbuf.at[b]], ssems[k],
                            add=True)
                        nxt = b + NBUF

                        @pl.when(nxt < HNB)
                        def _():
                            pltpu.make_async_copy(
                                rows.at[k], accum.at[sbuf.at[b]],
                                ssems[k]).wait()
                            pltpu.async_copy(
                                x_hbm.at[chunk].at[gbuf.at[nxt]],
                                rows.at[k], gsems[k])

                for k in range(NBUF):
                    pltpu.make_async_copy(
                        rows.at[k], accum.at[sbuf.at[HNB - NBUF + k]],
                        ssems[k]).wait()

            plsc.subcore_barrier()

            wnorm, wlast = wb_copy(jj, chunk)

            @pl.when(jnp.logical_not(last))
            def _():
                wnorm.start()

            @pl.when(last)
            def _():
                wlast.start()

        fcc, fjj = tasks[-1]
        wnorm, wlast = wb_copy(fjj, cid * cpc + fcc)

        @pl.when(jnp.logical_not(last))
        def _():
            wnorm.wait()

        @pl.when(last)
        def _():
            wlast.wait()

    mesh = plsc.VectorSubcoreMesh(core_axis_name="c", subcore_axis_name="s")
    return pl.kernel(
        body,
        out_type=jax.ShapeDtypeStruct((nj, num_chunks, N, LANE), jnp.float32),
        mesh=mesh,
        scratch_types=[
            pltpu.VMEM_SHARED((N, LANE), jnp.float32),
            pltpu.VMEM((HNB, B), jnp.int32),
            pltpu.VMEM((HNB, B), jnp.int32),
            pltpu.VMEM((NBUF, B, LANE), jnp.float32),
        ] + [pltpu.SemaphoreType.DMA] * (2 * NBUF + 1),
    )


def _tc_partial(num_chunks, nj, m_blk=2000):
    K = (nj + 1) * num_chunks
    grid = (N // m_blk, K)

    def body(agg_ref, x_ref, w_ref, out_ref, acc_ref):
        k = pl.program_id(1)

        @pl.when(k == 0)
        def _():
            acc_ref[...] = jnp.zeros_like(acc_ref)

        piece = jnp.where(k < nj * num_chunks, agg_ref[0, 0], x_ref[0])
        acc_ref[...] += jax.lax.dot(
            piece.astype(jnp.bfloat16), w_ref[0],
            preferred_element_type=jnp.float32)

        @pl.when(k == K - 1)
        def _():
            out_ref[...] = acc_ref[...]

    return pl.pallas_call(
        body,
        grid=grid,
        in_specs=[
            pl.BlockSpec((1, 1, m_blk, LANE),
                         lambda m, k: (jnp.minimum(k // num_chunks, nj - 1),
                                       k % num_chunks, m, 0)),
            pl.BlockSpec((1, m_blk, LANE), lambda m, k: (k % num_chunks, m, 0)),
            pl.BlockSpec((1, LANE, DH), lambda m, k: (k, 0, 0)),
        ],
        out_specs=pl.BlockSpec((m_blk, DH), lambda m, k: (m, 0)),
        out_shape=jax.ShapeDtypeStruct((N, DH), jnp.float32),
        scratch_shapes=[pltpu.VMEM((m_blk, DH), jnp.float32)],
        compiler_params=pltpu.CompilerParams(
            dimension_semantics=("parallel", "arbitrary")),
    )


def _tc_final(num_chunks, nj, m_blk=2000):
    K = nj * num_chunks
    grid = (N // m_blk, K)

    def body(agg_ref, p_ref, w_ref, bt_ref, g_ref, b_ref, out_ref, acc_ref):
        k = pl.program_id(1)

        @pl.when(k == 0)
        def _():
            acc_ref[...] = p_ref[...]

        acc_ref[...] += jax.lax.dot(
            agg_ref[0, 0].astype(jnp.bfloat16), w_ref[0],
            preferred_element_type=jnp.float32)

        @pl.when(k == K - 1)
        def _():
            y = acc_ref[...] + bt_ref[...]
            mu = jnp.mean(y, axis=-1, keepdims=True)
            var = jnp.mean((y - mu) ** 2, axis=-1, keepdims=True)
            out_ref[...] = ((y - mu) * jax.lax.rsqrt(var + 1e-5)
                            * g_ref[...] + b_ref[...])

    return pl.pallas_call(
        body,
        grid=grid,
        in_specs=[
            pl.BlockSpec((1, 1, m_blk, LANE),
                         lambda m, k: (k // num_chunks, k % num_chunks, m, 0)),
            pl.BlockSpec((m_blk, DH), lambda m, k: (m, 0)),
            pl.BlockSpec((1, LANE, DH), lambda m, k: (k, 0, 0)),
            pl.BlockSpec((1, DH), lambda m, k: (0, 0)),
            pl.BlockSpec((1, DH), lambda m, k: (0, 0)),
            pl.BlockSpec((1, DH), lambda m, k: (0, 0)),
        ],
        out_specs=pl.BlockSpec((m_blk, DH), lambda m, k: (m, 0)),
        out_shape=jax.ShapeDtypeStruct((N, DH), jnp.float32),
        scratch_shapes=[pltpu.VMEM((m_blk, DH), jnp.float32)],
        compiler_params=pltpu.CompilerParams(
            dimension_semantics=("parallel", "arbitrary")),
    )


def _chunked(h, num_chunks):
    return h.reshape(N, num_chunks, LANE).transpose(1, 0, 2)


def _wbig(w_in, w_out, w_lin, num_chunks):
    wcat = jnp.concatenate(
        [0.5 * w_in, 0.5 * w_out, jnp.sum(w_lin, axis=0)[None]], axis=0)
    d = num_chunks * LANE
    return wcat.reshape(9, num_chunks, LANE, DH).reshape(9 * num_chunks, LANE, DH)


def kernel(x, edge_index_forward, edge_index_onset, edge_index_sustain,
           edge_index_rest, W1_in, b1_in, W1_out, b1_out, W1_lin, b1_lin,
           W2_in, b2_in, W2_out, b2_out, W2_lin, b2_lin,
           ln1_g, ln1_b, ln2_g, ln2_b):
    eis = (edge_index_forward, edge_index_onset, edge_index_sustain,
           edge_index_rest)
    gidx = jnp.stack([e[0] for e in eis] + [e[1] for e in eis])
    sidx = jnp.stack([e[1] for e in eis] + [e[0] for e in eis])
    gidx = gidx.reshape(8, NT * NH, HNB, B)
    sidx = sidx.reshape(8, NT * NH, HNB, B)
    zrows = jnp.zeros((RPT_LAST, LANE), jnp.float32)

    NJA = 6

    def layer(h, num_chunks, w_in, b_in, w_out, b_out, w_lin, b_lin, g, b):
        h_r = _chunked(h, num_chunks)
        agg_a = _sc_aggregate(num_chunks, 0, NJA)(h_r, gidx, sidx, zrows)
        agg_b = _sc_aggregate(num_chunks, NJA, 8 - NJA)(h_r, gidx, sidx, zrows)
        w9 = _wbig(w_in, w_out, w_lin, num_chunks)
        wa = jnp.concatenate(
            [w9[:NJA * num_chunks], w9[8 * num_chunks:]],
            axis=0).astype(jnp.bfloat16)
        wf = w9[NJA * num_chunks:8 * num_chunks].astype(jnp.bfloat16)
        btot = jnp.sum(0.5 * b_in + 0.5 * b_out + b_lin, axis=0)[None]
        part = _tc_partial(num_chunks, NJA)(agg_a, h_r, wa)
        return _tc_final(num_chunks, 8 - NJA)(
            agg_b, part, wf, btot, g[None], b[None])

    h1 = layer(x, 2, W1_in, b1_in, W1_out, b1_out, W1_lin, b1_lin,
               ln1_g, ln1_b)
    h2 = layer(h1, 4, W2_in, b2_in, W2_out, b2_out, W2_lin, b2_lin,
               ln2_g, ln2_b)
    return h2


# speedup vs baseline: 7.1972x; 1.0051x over previous
import functools

import jax
import jax.numpy as jnp
from jax import lax
from jax.experimental import pallas as pl
from jax.experimental.pallas import tpu as pltpu
from jax.experimental.pallas import tpu_sc as plsc

N = 10000
E = 160000
DH = 512
LANE = 128
NT = 16
NCORE = 2
EPT = E // NT
B = 125
NB = EPT // B
NBUF = 2
NH = 2
HNB = NB // NH
RPT = 624
RPT_LAST = N - (NT - 1) * RPT


def _sc_aggregate(num_chunks, j0, nj):
    cpc = num_chunks // NCORE

    def body(x_hbm, g_hbm, s_hbm, z_hbm, out_hbm,
             accum, gbuf, sbuf, rows, *sems):
        cid = lax.axis_index("c")
        sid = lax.axis_index("s")
        row0 = sid * RPT

        last = sid == NT - 1
        gsems = sems[:NBUF]
        ssems = sems[NBUF:2 * NBUF]
        wbsem = sems[2 * NBUF]

        def wb_copy(jj, chunk):
            return (
                pltpu.make_async_copy(
                    accum.at[pl.ds(row0, RPT)],
                    out_hbm.at[jj, chunk, pl.ds(row0, RPT)], wbsem),
                pltpu.make_async_copy(
                    accum.at[pl.ds(row0, RPT_LAST)],
                    out_hbm.at[jj, chunk, pl.ds(row0, RPT_LAST)], wbsem),
            )

        tasks = [(cc, jj) for cc in range(cpc) for jj in range(nj)]
        for ti, (cc, jj) in enumerate(tasks):
            chunk = cid * cpc + cc
            j = j0 + jj

            pltpu.sync_copy(g_hbm.at[j, sid * NH], gbuf)
            pltpu.sync_copy(s_hbm.at[j, sid * NH], sbuf)
            for k in range(NBUF):
                pltpu.async_copy(
                    x_hbm.at[chunk].at[gbuf.at[k]], rows.at[k], gsems[k])

            if ti > 0:
                pcc, pjj = tasks[ti - 1]
                wnorm, wlast = wb_copy(pjj, cid * cpc + pcc)

                @pl.when(jnp.logical_not(last))
                def _():
                    wnorm.wait()

                @pl.when(last)
                def _():
                    wlast.wait()

            @pl.when(jnp.logical_not(last))
            def _():
                pltpu.sync_copy(z_hbm.at[pl.ds(0, RPT)],
                                accum.at[pl.ds(row0, RPT)])

            @pl.when(last)
            def _():
                pltpu.sync_copy(z_hbm, accum.at[pl.ds(row0, RPT_LAST)])

            plsc.subcore_barrier()

            for h in range(NH):
                if h > 0:
                    pltpu.sync_copy(g_hbm.at[j, sid * NH + h], gbuf)
                    pltpu.sync_copy(s_hbm.at[j, sid * NH + h], sbuf)
                    for k in range(NBUF):
                        pltpu.async_copy(
                            x_hbm.at[chunk].at[gbuf.at[k]], rows.at[k],
                            gsems[k])

                @pl.loop(0, HNB, step=NBUF)
                def _(g):
                    for k in range(NBUF):
                        b = g + k
                        pltpu.make_async_copy(
                            x_hbm.at[chunk].at[gbuf.at[b]], rows.at[k],
                            gsems[k]).wait()
                        pltpu.async_copy(
                            rows.at[k], accum.at[sbuf.at[b]], ssems[k],
                            add=True)
                        nxt = b + NBUF

                        @pl.when(nxt < HNB)
                        def _():
                            pltpu.make_async_copy(
                                rows.at[k], accum.at[sbuf.at[b]],
                                ssems[k]).wait()
                            pltpu.async_copy(
                                x_hbm.at[chunk].at[gbuf.at[nxt]],
                                rows.at[k], gsems[k])

                for k in range(NBUF):
                    pltpu.make_async_copy(
                        rows.at[k], accum.at[sbuf.at[HNB - NBUF + k]],
                        ssems[k]).wait()

            plsc.subcore_barrier()

            wnorm, wlast = wb_copy(jj, chunk)

            @pl.when(jnp.logical_not(last))
            def _():
                wnorm.start()

            @pl.when(last)
            def _():
                wlast.start()

        fcc, fjj = tasks[-1]
        wnorm, wlast = wb_copy(fjj, cid * cpc + fcc)

        @pl.when(jnp.logical_not(last))
        def _():
            wnorm.wait()

        @pl.when(last)
        def _():
            wlast.wait()

    mesh = plsc.VectorSubcoreMesh(core_axis_name="c", subcore_axis_name="s")
    return pl.kernel(
        body,
        out_type=jax.ShapeDtypeStruct((nj, num_chunks, N, LANE), jnp.float32),
        mesh=mesh,
        scratch_types=[
            pltpu.VMEM_SHARED((N, LANE), jnp.float32),
            pltpu.VMEM((HNB, B), jnp.int32),
            pltpu.VMEM((HNB, B), jnp.int32),
            pltpu.VMEM((NBUF, B, LANE), jnp.float32),
        ] + [pltpu.SemaphoreType.DMA] * (2 * NBUF + 1),
    )


def _tc_partial(num_chunks, nj, m_blk=2000):
    K = (nj + 1) * num_chunks
    grid = (N // m_blk, K)

    def body(agg_ref, x_ref, w_ref, out_ref, acc_ref):
        k = pl.program_id(1)

        @pl.when(k == 0)
        def _():
            acc_ref[...] = jnp.zeros_like(acc_ref)

        piece = jnp.where(k < nj * num_chunks, agg_ref[0, 0], x_ref[0])
        acc_ref[...] += jax.lax.dot(
            piece.astype(jnp.bfloat16), w_ref[0],
            preferred_element_type=jnp.float32)

        @pl.when(k == K - 1)
        def _():
            out_ref[...] = acc_ref[...]

    return pl.pallas_call(
        body,
        grid=grid,
        in_specs=[
            pl.BlockSpec((1, 1, m_blk, LANE),
                         lambda m, k: (jnp.minimum(k // num_chunks, nj - 1),
                                       k % num_chunks, m, 0)),
            pl.BlockSpec((1, m_blk, LANE), lambda m, k: (k % num_chunks, m, 0)),
            pl.BlockSpec((1, LANE, DH), lambda m, k: (k, 0, 0)),
        ],
        out_specs=pl.BlockSpec((m_blk, DH), lambda m, k: (m, 0)),
        out_shape=jax.ShapeDtypeStruct((N, DH), jnp.float32),
        scratch_shapes=[pltpu.VMEM((m_blk, DH), jnp.float32)],
        compiler_params=pltpu.CompilerParams(
            dimension_semantics=("parallel", "arbitrary")),
    )


def _tc_final(num_chunks, nj, m_blk=2000):
    K = nj * num_chunks
    grid = (N // m_blk, K)

    def body(agg_ref, p_ref, w_ref, bt_ref, g_ref, b_ref, out_ref, acc_ref):
        k = pl.program_id(1)

        @pl.when(k == 0)
        def _():
            acc_ref[...] = p_ref[...]

        acc_ref[...] += jax.lax.dot(
            agg_ref[0, 0].astype(jnp.bfloat16), w_ref[0],
            preferred_element_type=jnp.float32)

        @pl.when(k == K - 1)
        def _():
            y = acc_ref[...] + bt_ref[...]
            mu = jnp.mean(y, axis=-1, keepdims=True)
            var = jnp.mean((y - mu) ** 2, axis=-1, keepdims=True)
            out_ref[...] = ((y - mu) * jax.lax.rsqrt(var + 1e-5)
                            * g_ref[...] + b_ref[...])

    return pl.pallas_call(
        body,
        grid=grid,
        in_specs=[
            pl.BlockSpec((1, 1, m_blk, LANE),
                         lambda m, k: (k // num_chunks, k % num_chunks, m, 0)),
            pl.BlockSpec((m_blk, DH), lambda m, k: (m, 0)),
            pl.BlockSpec((1, LANE, DH), lambda m, k: (k, 0, 0)),
            pl.BlockSpec((1, DH), lambda m, k: (0, 0)),
            pl.BlockSpec((1, DH), lambda m, k: (0, 0)),
            pl.BlockSpec((1, DH), lambda m, k: (0, 0)),
        ],
        out_specs=pl.BlockSpec((m_blk, DH), lambda m, k: (m, 0)),
        out_shape=jax.ShapeDtypeStruct((N, DH), jnp.float32),
        scratch_shapes=[pltpu.VMEM((m_blk, DH), jnp.float32)],
        compiler_params=pltpu.CompilerParams(
            dimension_semantics=("parallel", "arbitrary")),
    )


def _chunked(h, num_chunks):
    return h.reshape(N, num_chunks, LANE).transpose(1, 0, 2)


def _wbig(w_in, w_out, w_lin, num_chunks):
    wcat = jnp.concatenate(
        [0.5 * w_in, 0.5 * w_out, jnp.sum(w_lin, axis=0)[None]], axis=0)
    d = num_chunks * LANE
    return wcat.reshape(9, num_chunks, LANE, DH).reshape(9 * num_chunks, LANE, DH)


def kernel(x, edge_index_forward, edge_index_onset, edge_index_sustain,
           edge_index_rest, W1_in, b1_in, W1_out, b1_out, W1_lin, b1_lin,
           W2_in, b2_in, W2_out, b2_out, W2_lin, b2_lin,
           ln1_g, ln1_b, ln2_g, ln2_b):
    eis = (edge_index_forward, edge_index_onset, edge_index_sustain,
           edge_index_rest)
    gidx = jnp.stack([e[0] for e in eis] + [e[1] for e in eis])
    sidx = jnp.stack([e[1] for e in eis] + [e[0] for e in eis])
    gidx = gidx.reshape(8, NT * NH, HNB, B)
    sidx = sidx.reshape(8, NT * NH, HNB, B)
    zrows = jnp.zeros((RPT_LAST, LANE), jnp.float32)

    NJA = 7

    def layer(h, num_chunks, w_in, b_in, w_out, b_out, w_lin, b_lin, g, b):
        h_r = _chunked(h, num_chunks)
        agg_a = _sc_aggregate(num_chunks, 0, NJA)(h_r, gidx, sidx, zrows)
        agg_b = _sc_aggregate(num_chunks, NJA, 8 - NJA)(h_r, gidx, sidx, zrows)
        w9 = _wbig(w_in, w_out, w_lin, num_chunks)
        wa = jnp.concatenate(
            [w9[:NJA * num_chunks], w9[8 * num_chunks:]],
            axis=0).astype(jnp.bfloat16)
        wf = w9[NJA * num_chunks:8 * num_chunks].astype(jnp.bfloat16)
        btot = jnp.sum(0.5 * b_in + 0.5 * b_out + b_lin, axis=0)[None]
        part = _tc_partial(num_chunks, NJA)(agg_a, h_r, wa)
        return _tc_final(num_chunks, 8 - NJA)(
            agg_b, part, wf, btot, g[None], b[None])

    h1 = layer(x, 2, W1_in, b1_in, W1_out, b1_out, W1_lin, b1_lin,
               ln1_g, ln1_b)
    h2 = layer(h1, 4, W2_in, b2_in, W2_out, b2_out, W2_lin, b2_lin,
               ln2_g, ln2_b)
    return h2


# speedup vs baseline: 7.3001x; 1.0143x over previous
import functools

import jax
import jax.numpy as jnp
from jax import lax
from jax.experimental import pallas as pl
from jax.experimental.pallas import tpu as pltpu
from jax.experimental.pallas import tpu_sc as plsc

N = 10000
E = 160000
DH = 512
LANE = 128
NT = 16
NCORE = 2
EPT = E // NT
B = 125
NB = EPT // B
NBUF = 2
NH = 2
HNB = NB // NH
RPT = 624
RPT_LAST = N - (NT - 1) * RPT


def _sc_aggregate(num_chunks, j0, nj):
    cpc = num_chunks // NCORE

    def body(x_hbm, g_hbm, s_hbm, z_hbm, out_hbm,
             accum, gbuf, sbuf, rows, *sems):
        cid = lax.axis_index("c")
        sid = lax.axis_index("s")
        row0 = sid * RPT

        last = sid == NT - 1
        gsems = sems[:NBUF]
        ssems = sems[NBUF:2 * NBUF]
        wbsem = sems[2 * NBUF]

        def wb_copy(jj, chunk):
            return (
                pltpu.make_async_copy(
                    accum.at[pl.ds(row0, RPT)],
                    out_hbm.at[jj, chunk, pl.ds(row0, RPT)], wbsem),
                pltpu.make_async_copy(
                    accum.at[pl.ds(row0, RPT_LAST)],
                    out_hbm.at[jj, chunk, pl.ds(row0, RPT_LAST)], wbsem),
            )

        tasks = [(cc, jj) for cc in range(cpc) for jj in range(nj)]
        for ti, (cc, jj) in enumerate(tasks):
            chunk = cid * cpc + cc
            j = j0 + jj

            pltpu.sync_copy(g_hbm.at[j, sid * NH], gbuf)
            pltpu.sync_copy(s_hbm.at[j, sid * NH], sbuf)
            for k in range(NBUF):
                pltpu.async_copy(
                    x_hbm.at[chunk].at[gbuf.at[k]], rows.at[k], gsems[k])

            if ti > 0:
                pcc, pjj = tasks[ti - 1]
                wnorm, wlast = wb_copy(pjj, cid * cpc + pcc)

                @pl.when(jnp.logical_not(last))
                def _():
                    wnorm.wait()

                @pl.when(last)
                def _():
                    wlast.wait()

            @pl.when(jnp.logical_not(last))
            def _():
                pltpu.sync_copy(z_hbm.at[pl.ds(0, RPT)],
                                accum.at[pl.ds(row0, RPT)])

            @pl.when(last)
            def _():
                pltpu.sync_copy(z_hbm, accum.at[pl.ds(row0, RPT_LAST)])

            plsc.subcore_barrier()

            for h in range(NH):
                if h > 0:
                    pltpu.sync_copy(g_hbm.at[j, sid * NH + h], gbuf)
                    pltpu.sync_copy(s_hbm.at[j, sid * NH + h], sbuf)
                    for k in range(NBUF):
                        pltpu.async_copy(
                            x_hbm.at[chunk].at[gbuf.at[k]], rows.at[k],
                            gsems[k])

                @pl.loop(0, HNB, step=NBUF)
                def _(g):
                    for k in range(NBUF):
                        b = g + k
                        pltpu.make_async_copy(
                            x_hbm.at[chunk].at[gbuf.at[b]], rows.at[k],
                            gsems[k]).wait()
                        pltpu.async_copy(
                            rows.at[k], accum.at[sbuf.at[b]], ssems[k],
                            add=True)
                        nxt = b + NBUF

                        @pl.when(nxt < HNB)
                        def _():
                            pltpu.make_async_copy(
                                rows.at[k], accum.at[sbuf.at[b]],
                                ssems[k]).wait()
                            pltpu.async_copy(
                                x_hbm.at[chunk].at[gbuf.at[nxt]],
                                rows.at[k], gsems[k])

                for k in range(NBUF):
                    pltpu.make_async_copy(
                        rows.at[k], accum.at[sbuf.at[HNB - NBUF + k]],
                        ssems[k]).wait()

            plsc.subcore_barrier()

            wnorm, wlast = wb_copy(jj, chunk)

            @pl.when(jnp.logical_not(last))
            def _():
                wnorm.start()

            @pl.when(last)
            def _():
                wlast.start()

        fcc, fjj = tasks[-1]
        wnorm, wlast = wb_copy(fjj, cid * cpc + fcc)

        @pl.when(jnp.logical_not(last))
        def _():
            wnorm.wait()

        @pl.when(last)
        def _():
            wlast.wait()

    mesh = plsc.VectorSubcoreMesh(core_axis_name="c", subcore_axis_name="s")
    return pl.kernel(
        body,
        out_type=jax.ShapeDtypeStruct((nj, num_chunks, N, LANE), jnp.float32),
        mesh=mesh,
        scratch_types=[
            pltpu.VMEM_SHARED((N, LANE), jnp.float32),
            pltpu.VMEM((HNB, B), jnp.int32),
            pltpu.VMEM((HNB, B), jnp.int32),
            pltpu.VMEM((NBUF, B, LANE), jnp.float32),
        ] + [pltpu.SemaphoreType.DMA] * (2 * NBUF + 1),
    )


def _tc_partial(num_chunks, nj, m_blk=2000):
    K = (nj + 1) * num_chunks
    grid = (N // m_blk, K)

    def body(agg_ref, x_ref, w_ref, out_ref, acc_ref):
        k = pl.program_id(1)

        @pl.when(k == 0)
        def _():
            acc_ref[...] = jnp.zeros_like(acc_ref)

        piece = jnp.where(k < nj * num_chunks, agg_ref[0, 0], x_ref[0])
        acc_ref[...] += jax.lax.dot(
            piece.astype(jnp.bfloat16), w_ref[0],
            preferred_element_type=jnp.float32)

        @pl.when(k == K - 1)
        def _():
            out_ref[...] = acc_ref[...]

    return pl.pallas_call(
        body,
        grid=grid,
        in_specs=[
            pl.BlockSpec((1, 1, m_blk, LANE),
                         lambda m, k: (jnp.minimum(k // num_chunks, nj - 1),
                                       k % num_chunks, m, 0)),
            pl.BlockSpec((1, m_blk, LANE), lambda m, k: (k % num_chunks, m, 0)),
            pl.BlockSpec((1, LANE, DH), lambda m, k: (k, 0, 0)),
        ],
        out_specs=pl.BlockSpec((m_blk, DH), lambda m, k: (m, 0)),
        out_shape=jax.ShapeDtypeStruct((N, DH), jnp.float32),
        scratch_shapes=[pltpu.VMEM((m_blk, DH), jnp.float32)],
        compiler_params=pltpu.CompilerParams(
            dimension_semantics=("parallel", "arbitrary")),
    )


def _tc_final(num_chunks, nj, out_chunks=None, m_blk=2000):
    K = nj * num_chunks
    grid = (N // m_blk, K)

    def body(agg_ref, p_ref, w_ref, bt_ref, g_ref, b_ref, out_ref, acc_ref):
        k = pl.program_id(1)

        @pl.when(k == 0)
        def _():
            acc_ref[...] = p_ref[...]

        acc_ref[...] += jax.lax.dot(
            agg_ref[0, 0].astype(jnp.bfloat16), w_ref[0],
            preferred_element_type=jnp.float32)

        @pl.when(k == K - 1)
        def _():
            y = acc_ref[...] + bt_ref[...]
            mu = jnp.mean(y, axis=-1, keepdims=True)
            var = jnp.mean((y - mu) ** 2, axis=-1, keepdims=True)
            y = ((y - mu) * jax.lax.rsqrt(var + 1e-5)
                 * g_ref[...] + b_ref[...])
            if out_chunks is None:
                out_ref[...] = y
            else:
                out_ref[...] = jnp.swapaxes(
                    y.reshape(m_blk, out_chunks, LANE), 0, 1)

    if out_chunks is None:
        out_spec = pl.BlockSpec((m_blk, DH), lambda m, k: (m, 0))
        out_shape = jax.ShapeDtypeStruct((N, DH), jnp.float32)
    else:
        out_spec = pl.BlockSpec((out_chunks, m_blk, LANE),
                                lambda m, k: (0, m, 0))
        out_shape = jax.ShapeDtypeStruct((out_chunks, N, LANE), jnp.float32)

    return pl.pallas_call(
        body,
        grid=grid,
        in_specs=[
            pl.BlockSpec((1, 1, m_blk, LANE),
                         lambda m, k: (k // num_chunks, k % num_chunks, m, 0)),
            pl.BlockSpec((m_blk, DH), lambda m, k: (m, 0)),
            pl.BlockSpec((1, LANE, DH), lambda m, k: (k, 0, 0)),
            pl.BlockSpec((1, DH), lambda m, k: (0, 0)),
            pl.BlockSpec((1, DH), lambda m, k: (0, 0)),
            pl.BlockSpec((1, DH), lambda m, k: (0, 0)),
        ],
        out_specs=out_spec,
        out_shape=out_shape,
        scratch_shapes=[pltpu.VMEM((m_blk, DH), jnp.float32)],
        compiler_params=pltpu.CompilerParams(
            dimension_semantics=("parallel", "arbitrary")),
    )


def _chunked(h, num_chunks):
    return h.reshape(N, num_chunks, LANE).transpose(1, 0, 2)


def _wbig(w_in, w_out, w_lin, num_chunks):
    wcat = jnp.concatenate(
        [0.5 * w_in, 0.5 * w_out, jnp.sum(w_lin, axis=0)[None]], axis=0)
    d = num_chunks * LANE
    return wcat.reshape(9, num_chunks, LANE, DH).reshape(9 * num_chunks, LANE, DH)


def kernel(x, edge_index_forward, edge_index_onset, edge_index_sustain,
           edge_index_rest, W1_in, b1_in, W1_out, b1_out, W1_lin, b1_lin,
           W2_in, b2_in, W2_out, b2_out, W2_lin, b2_lin,
           ln1_g, ln1_b, ln2_g, ln2_b):
    eis = (edge_index_forward, edge_index_onset, edge_index_sustain,
           edge_index_rest)
    gidx = jnp.stack([e[0] for e in eis] + [e[1] for e in eis])
    sidx = jnp.stack([e[1] for e in eis] + [e[0] for e in eis])
    gidx = gidx.reshape(8, NT * NH, HNB, B)
    sidx = sidx.reshape(8, NT * NH, HNB, B)
    zrows = jnp.zeros((RPT_LAST, LANE), jnp.float32)

    NJA = 7

    def layer(h_r, num_chunks, w_in, b_in, w_out, b_out, w_lin, b_lin, g, b,
              out_chunks=None):
        agg_a = _sc_aggregate(num_chunks, 0, NJA)(h_r, gidx, sidx, zrows)
        agg_b = _sc_aggregate(num_chunks, NJA, 8 - NJA)(h_r, gidx, sidx, zrows)
        w9 = _wbig(w_in, w_out, w_lin, num_chunks)
        wa = jnp.concatenate(
            [w9[:NJA * num_chunks], w9[8 * num_chunks:]],
            axis=0).astype(jnp.bfloat16)
        wf = w9[NJA * num_chunks:8 * num_chunks].astype(jnp.bfloat16)
        btot = jnp.sum(0.5 * b_in + 0.5 * b_out + b_lin, axis=0)[None]
        part = _tc_partial(num_chunks, NJA)(agg_a, h_r, wa)
        return _tc_final(num_chunks, 8 - NJA, out_chunks)(
            agg_b, part, wf, btot, g[None], b[None])

    h1_r = layer(_chunked(x, 2), 2, W1_in, b1_in, W1_out, b1_out,
                 W1_lin, b1_lin, ln1_g, ln1_b, out_chunks=4)
    h2 = layer(h1_r, 4, W2_in, b2_in, W2_out, b2_out, W2_lin, b2_lin,
               ln2_g, ln2_b)
    return h2
